# baseline (speedup 1.0000x reference)
import functools

import jax
import jax.numpy as jnp
from jax import lax
from jax.experimental import pallas as pl
from jax.experimental.pallas import tpu as pltpu
from jax.experimental.pallas import tpu_sc as plsc

B = 16384
D = 32
NC, NS = 2, 16
NW = NC * NS
BPW = B // NW
CH = 128
NCH = BPW // CH


@functools.cache
def _make_sc_gather():
    mesh = plsc.VectorSubcoreMesh(core_axis_name="c", subcore_axis_name="s")

    @functools.partial(
        pl.kernel,
        out_type=[jax.ShapeDtypeStruct((B, D), jnp.float32)] * 4,
        mesh=mesh,
        compiler_params=pltpu.CompilerParams(use_tc_tiling_on_sc=False),
        scratch_types=[
            pltpu.VMEM((BPW,), jnp.int32),
            pltpu.VMEM((BPW,), jnp.int32),
            pltpu.VMEM((BPW, D), jnp.float32),
            pltpu.VMEM((BPW, D), jnp.float32),
            pltpu.VMEM((BPW, D), jnp.float32),
            pltpu.VMEM((BPW, D), jnp.float32),
            pltpu.SemaphoreType.DMA,
            pltpu.SemaphoreType.DMA,
            pltpu.SemaphoreType.DMA,
            pltpu.SemaphoreType.DMA,
            pltpu.SemaphoreType.DMA,
        ],
    )
    def sc_gather(uidx_hbm, iidx_hbm, gu_hbm, gi_hbm, mu_hbm, mi_hbm,
                  gu_out, gi_out, mu_out, mi_out,
                  uidx_v, iidx_v, gu_v, gi_v, mu_v, mi_v,
                  sem0, sem1, sem2, sem3, ssem):
        wid = lax.axis_index("s") * NC + lax.axis_index("c")
        base = wid * BPW
        pltpu.sync_copy(uidx_hbm.at[pl.ds(base, BPW)], uidx_v)
        pltpu.sync_copy(iidx_hbm.at[pl.ds(base, BPW)], iidx_v)
        plan = (
            (gu_hbm, uidx_v, gu_v, sem0, gu_out),
            (gi_hbm, iidx_v, gi_v, sem1, gi_out),
            (mu_hbm, uidx_v, mu_v, sem2, mu_out),
            (mi_hbm, iidx_v, mi_v, sem3, mi_out),
        )
        copies = []
        for tbl, idx_v, dst, sem, _ in plan:
            for j in range(NCH):
                copies.append(pltpu.async_copy(
                    tbl.at[idx_v.at[pl.ds(j * CH, CH)]],
                    dst.at[pl.ds(j * CH, CH)], sem))
        stores = []
        for t, (_, _, dst, _, out) in enumerate(plan):
            for j in range(NCH):
                copies[t * NCH + j].wait()
            stores.append(pltpu.async_copy(dst, out.at[pl.ds(base, BPW)], ssem))
        for st in stores:
            st.wait()

    return sc_gather


BLK = 2048


def _mlp_body(gu_ref, gi_ref, mu_ref, mi_ref, w1_ref, b1_ref, w2_ref, b2_ref,
              w3_ref, b3_ref, wpg_ref, wph_ref, bp_ref, out_ref):
    gmf = gu_ref[...] * gi_ref[...]
    x = jnp.concatenate([mu_ref[...], mi_ref[...]], axis=1)
    h = jnp.maximum(jnp.dot(x, w1_ref[...],
                            preferred_element_type=jnp.float32) + b1_ref[...], 0.0)
    h = jnp.maximum(jnp.dot(h, w2_ref[...],
                            preferred_element_type=jnp.float32) + b2_ref[...], 0.0)
    h = jnp.maximum(jnp.dot(h, w3_ref[...],
                            preferred_element_type=jnp.float32) + b3_ref[...], 0.0)
    logit = (jnp.dot(gmf, wpg_ref[...], preferred_element_type=jnp.float32)
             + jnp.dot(h, wph_ref[...], preferred_element_type=jnp.float32)
             + bp_ref[0, 0])
    out_ref[...] = jax.nn.sigmoid(logit)


def _run_mlp(gu, gi, mu, mi, W1, b1, W2, b2, W3, b3, Wpg, Wph, bp):
    grid = (B // BLK,)
    row_spec = pl.BlockSpec((BLK, D), lambda i: (i, 0))

    def whole(shape):
        return pl.BlockSpec(shape, lambda i: (0,) * len(shape))

    out = pl.pallas_call(
        _mlp_body,
        grid=grid,
        in_specs=[
            row_spec, row_spec, row_spec, row_spec,
            whole((64, 32)), whole((1, 32)),
            whole((32, 16)), whole((1, 16)),
            whole((16, 8)), whole((1, 8)),
            whole((32, 1)), whole((8, 1)), whole((1, 1)),
        ],
        out_specs=pl.BlockSpec((BLK, 1), lambda i: (i, 0)),
        out_shape=jax.ShapeDtypeStruct((B, 1), jnp.float32),
    )(gu, gi, mu, mi, W1, b1.reshape(1, 32), W2, b2.reshape(1, 16),
      W3, b3.reshape(1, 8), Wpg, Wph, bp.reshape(1, 1))
    return out.reshape(B)


def kernel(user_idx, item_idx, gmf_user, gmf_item, mlp_user, mlp_item,
           W1, b1, W2, b2, W3, b3, Wp, bp):
    uidx = user_idx.astype(jnp.int32)
    iidx = item_idx.astype(jnp.int32)
    gu, gi, mu, mi = _make_sc_gather()(uidx, iidx, gmf_user, gmf_item,
                                       mlp_user, mlp_item)
    Wpg = Wp[:D]
    Wph = Wp[D:]
    return _run_mlp(gu, gi, mu, mi, W1, b1, W2, b2, W3, b3, Wpg, Wph, bp)
